# Initial kernel scaffold; baseline (speedup 1.0000x reference)
import jax
import jax.numpy as jnp
from jax.experimental import pallas as pl


def kernel(x, edge_index, W, b):
    raise NotImplementedError("write your pallas kernel here")


# speedup vs baseline: 3.3703x; 3.3703x over previous
import functools

import jax
import jax.numpy as jnp
from jax import lax
from jax.experimental import pallas as pl
from jax.experimental.pallas import tpu as pltpu
from jax.experimental.pallas import tpu_sc as plsc

NC = 2
NS = 16
CHUNK = 128
AUG = 16
IDX_BLK = 40


def _sc_aggregate(x_aug, gidx, sidx, zeros_init, n_rows, n_chunks, da):
  rows_per_tile = n_rows // NS
  mesh = plsc.VectorSubcoreMesh(
      core_axis_name="c", subcore_axis_name="s", num_cores=NC, num_subcores=NS)

  @functools.partial(
      pl.kernel,
      out_type=jax.ShapeDtypeStruct((NC, n_rows, da), jnp.float32),
      mesh=mesh,
      compiler_params=pltpu.CompilerParams(use_tc_tiling_on_sc=False),
      scratch_types=[
          pltpu.VMEM((IDX_BLK, CHUNK), jnp.int32),
          pltpu.VMEM((IDX_BLK, CHUNK), jnp.int32),
          pltpu.VMEM((CHUNK, da), jnp.float32),
          pltpu.VMEM_SHARED((n_rows, da), jnp.float32),
          pltpu.SemaphoreType.DMA,
      ],
  )
  def agg(x_hbm, g_hbm, s_hbm, z_hbm, out_hbm, g_v, s_v, rows_v, acc, sem):
    c = lax.axis_index("c")
    s = lax.axis_index("s")
    r0 = s * rows_per_tile
    pltpu.sync_copy(z_hbm, acc.at[pl.ds(r0, rows_per_tile), :])
    plsc.subcore_barrier()

    def blk_body(bi, carry):
      pltpu.sync_copy(g_hbm.at[c, s, pl.ds(bi * IDX_BLK, IDX_BLK)], g_v)
      pltpu.sync_copy(s_hbm.at[c, s, pl.ds(bi * IDX_BLK, IDX_BLK)], s_v)

      def body(j, carry2):
        pltpu.async_copy(x_hbm.at[g_v.at[j]], rows_v, sem).wait()
        pltpu.sync_copy(rows_v, acc.at[s_v.at[j]], add=True)
        return carry2

      return lax.fori_loop(0, IDX_BLK, body, carry)

    lax.fori_loop(0, n_chunks // IDX_BLK, blk_body, 0)
    plsc.subcore_barrier()
    pltpu.sync_copy(acc.at[pl.ds(r0, rows_per_tile), :],
                    out_hbm.at[c, pl.ds(r0, rows_per_tile), :])

  return agg(x_aug, gidx, sidx, zeros_init)


def _tc_combine(x, acc, wt, b2, d_in, d_out, da):
  n = x.shape[0]
  blk = 1000
  grid = (n // blk,)

  def body(x_ref, ai_ref, ao_ref, w_ref, b_ref, o_ref):
    xb = x_ref[...]
    ai = ai_ref[0]
    ao = ao_ref[0]
    mi = ai[:, :d_in] / jnp.maximum(ai[:, d_in:d_in + 1], 1.0)
    mo = ao[:, :d_in] / jnp.maximum(ao[:, d_in:d_in + 1], 1.0)
    w = w_ref[...]
    o = (jnp.dot(xb, w[:d_in], preferred_element_type=jnp.float32)
         + jnp.dot(mi, w[d_in:2 * d_in], preferred_element_type=jnp.float32)
         + jnp.dot(mo, w[2 * d_in:3 * d_in], preferred_element_type=jnp.float32))
    o_ref[...] = jnp.maximum(o + b_ref[...], 0.0)

  return pl.pallas_call(
      body,
      grid=grid,
      in_specs=[
          pl.BlockSpec((blk, d_in), lambda i: (i, 0)),
          pl.BlockSpec((1, blk, da), lambda i: (0, i, 0)),
          pl.BlockSpec((1, blk, da), lambda i: (1, i, 0)),
          pl.BlockSpec((3 * d_in, d_out), lambda i: (0, 0)),
          pl.BlockSpec((1, d_out), lambda i: (0, 0)),
      ],
      out_specs=pl.BlockSpec((blk, d_out), lambda i: (i, 0)),
      out_shape=jax.ShapeDtypeStruct((n, d_out), jnp.float32),
  )(x, acc, acc, wt, b2)


def kernel(x, edge_index, W, b):
  n, d_in = x.shape
  d_out = W.shape[0]
  da = d_in + AUG
  e = edge_index.shape[1]

  src = edge_index[0].astype(jnp.int32)
  dst = edge_index[1].astype(jnp.int32)

  chunks = -(-e // (NS * CHUNK))
  per_tile_chunks = -(-chunks // IDX_BLK) * IDX_BLK
  e_pad = per_tile_chunks * NS * CHUNK
  pad = e_pad - e
  g0 = jnp.pad(src, (0, pad))
  g1 = jnp.pad(dst, (0, pad))
  s0 = jnp.pad(dst, (0, pad), constant_values=n)
  s1 = jnp.pad(src, (0, pad), constant_values=n)
  gidx = jnp.stack([g0, g1]).reshape(NC, NS, per_tile_chunks, CHUNK)
  sidx = jnp.stack([s0, s1]).reshape(NC, NS, per_tile_chunks, CHUNK)

  n_rows = -(-(n + 1) // (NS * 8)) * NS * 8

  x_aug = jnp.concatenate(
      [x, jnp.ones((n, 1), jnp.float32), jnp.zeros((n, AUG - 1), jnp.float32)],
      axis=1)
  zeros_init = jnp.zeros((n_rows // NS, da), jnp.float32)

  acc = _sc_aggregate(x_aug, gidx, sidx, zeros_init, n_rows, per_tile_chunks, da)

  wt = W.T
  b2 = b.reshape(1, d_out)
  return _tc_combine(x, acc, wt, b2, d_in, d_out, da)
